# baseline (speedup 1.0000x reference)
import functools

import jax
import jax.numpy as jnp
from jax import lax
from jax.experimental import pallas as pl
from jax.experimental.pallas import tpu as pltpu
from jax.experimental.pallas import tpu_sc as plsc

B = 4096
L = 50
VOCAB = 1000000
D = 32
LP = L + 2

NC = 2
NS = 16
NW = NC * NS
BT = B // 128

_mesh = plsc.VectorSubcoreMesh(core_axis_name="c", subcore_axis_name="s")


@functools.partial(
    pl.kernel,
    out_type=jax.ShapeDtypeStruct((LP, D // 8, BT, 8, 128), jnp.float32),
    mesh=_mesh,
    scratch_types=[
        pltpu.VMEM((L, 128), jnp.int32),
        pltpu.VMEM((2, 128), jnp.int32),
        pltpu.VMEM((2, 128, D), jnp.float32),
        pltpu.VMEM((2, D // 8, 8, 128), jnp.float32),
        pltpu.SemaphoreType.DMA,
        pltpu.SemaphoreType.DMA,
    ],
    compiler_params=pltpu.CompilerParams(
        use_tc_tiling_on_sc=False, needs_layout_passes=False
    ),
)
def _emb_lookup(idxt_hbm, table_hbm, out_hbm, idx_all, idxq_v, rows_v, t_v,
                gsem, wsem):
    j = lax.axis_index("s") * NC + lax.axis_index("c")
    lanes = lax.iota(jnp.int32, 16)
    zv = jnp.zeros((16,), jnp.float32)

    def fire_gather(slot, l):
        for k in range(8):
            v = idx_all[l, 16 * k:16 * (k + 1)]
            idxq_v[slot, 16 * k:16 * (k + 1)] = (
                ((v >> 13) << 13) | ((v & 2047) << 2) | ((v >> 11) & 3)
            )
        pltpu.async_copy(
            table_hbm.at[idxq_v.at[slot]], rows_v.at[slot], gsem
        )

    def wait_gather(slot):
        pltpu.make_async_copy(
            table_hbm.at[pl.ds(0, 128), :], rows_v.at[slot], gsem
        ).wait()

    def fire_writes(slot, lo):
        for i in range(D // 8):
            pltpu.async_copy(t_v.at[slot, i], out_hbm.at[lo, i, j], wsem)

    def drain_writes(slot):
        for i in range(D // 8):
            pltpu.make_async_copy(
                t_v.at[slot, i], out_hbm.at[0, i, j], wsem
            ).wait()

    def transpose(gslot, wslot, l):
        for k in range(8):
            row = 16 * k + lanes
            for i in range(D // 8):
                for s in range(8):
                    col = jnp.full((16,), 8 * i + s, jnp.int32)
                    t_v[wslot, i, s, 16 * k:16 * (k + 1)] = plsc.load_gather(
                        rows_v.at[gslot], [row, col]
                    )

    for slot in range(2):
        for i in range(D // 8):
            for s in range(8):
                for k in range(8):
                    t_v[slot, i, s, 16 * k:16 * (k + 1)] = zv

    pltpu.sync_copy(idxt_hbm.at[:, pl.ds(128 * j, 128)], idx_all)

    fire_writes(0, 0)
    fire_writes(1, LP - 1)
    fire_gather(0, 0)
    fire_gather(1, 1)

    @pl.loop(0, L, step=2)
    def _pos(base):
        for slot in range(2):
            wait_gather(slot)
            drain_writes(slot)
            transpose(slot, slot, base + slot)
            fire_writes(slot, base + slot + 1)

            @pl.when(base + slot + 2 < L)
            def _prefetch():
                fire_gather(slot, base + slot + 2)

    drain_writes(0)
    drain_writes(1)


_CB = 8192
_NBLK = (VOCAB + _CB - 1) // _CB
_RPB = _CB // 4


def _tc_transpose_body(in_ref, out_ref):
    x = in_ref[...]
    y = x.T
    out_ref[...] = jnp.concatenate(
        [y[q * _RPB:(q + 1) * _RPB, :] for q in range(4)], axis=1
    )


_tc_transpose = pl.pallas_call(
    _tc_transpose_body,
    out_shape=jax.ShapeDtypeStruct((_NBLK * _RPB, 4 * D), jnp.float32),
    grid=(_NBLK,),
    in_specs=[pl.BlockSpec((D, _CB), lambda g: (0, g))],
    out_specs=pl.BlockSpec((_RPB, 4 * D), lambda g: (g, 0)),
)


def kernel(indices, table):
    idxt = indices.astype(jnp.int32).T
    table128 = _tc_transpose(table.T)
    table32 = table128.reshape(_NBLK * _CB, D)
    out5 = _emb_lookup(idxt, table32)
    val_emb = out5.transpose(2, 4, 0, 1, 3).reshape(B, LP, D)
    val_len = jnp.full((B,), LP, dtype=jnp.int64)
    return (val_emb, val_len)


# speedup vs baseline: 1.0007x; 1.0007x over previous
import functools

import jax
import jax.numpy as jnp
from jax import lax
from jax.experimental import pallas as pl
from jax.experimental.pallas import tpu as pltpu
from jax.experimental.pallas import tpu_sc as plsc

B = 4096
L = 50
VOCAB = 1000000
D = 32
LP = L + 2

NC = 2
NS = 16
NW = NC * NS
BT = B // 128

_mesh = plsc.VectorSubcoreMesh(core_axis_name="c", subcore_axis_name="s")


@functools.partial(
    pl.kernel,
    out_type=jax.ShapeDtypeStruct((LP, D // 8, BT, 8, 128), jnp.float32),
    mesh=_mesh,
    scratch_types=[
        pltpu.VMEM((L, 128), jnp.int32),
        pltpu.VMEM((3, 128), jnp.int32),
        pltpu.VMEM((3, 128, D), jnp.float32),
        pltpu.VMEM((2, D // 8, 8, 128), jnp.float32),
        pltpu.SemaphoreType.DMA,
        pltpu.SemaphoreType.DMA,
    ],
    compiler_params=pltpu.CompilerParams(
        use_tc_tiling_on_sc=False, needs_layout_passes=False
    ),
)
def _emb_lookup(idxt_hbm, table_hbm, out_hbm, idx_all, idxq_v, rows_v, t_v,
                gsem, wsem):
    j = lax.axis_index("s") * NC + lax.axis_index("c")
    lanes = lax.iota(jnp.int32, 16)
    zv = jnp.zeros((16,), jnp.float32)

    def fire_gather(slot, l):
        for k in range(8):
            v = idx_all[l, 16 * k:16 * (k + 1)]
            idxq_v[slot, 16 * k:16 * (k + 1)] = (
                ((v >> 13) << 13) | ((v & 2047) << 2) | ((v >> 11) & 3)
            )
        pltpu.async_copy(
            table_hbm.at[idxq_v.at[slot]], rows_v.at[slot], gsem
        )

    def wait_gather(slot):
        pltpu.make_async_copy(
            table_hbm.at[pl.ds(0, 128), :], rows_v.at[slot], gsem
        ).wait()

    def fire_writes(slot, lo):
        for i in range(D // 8):
            pltpu.async_copy(t_v.at[slot, i], out_hbm.at[lo, i, j], wsem)

    def drain_writes(slot):
        for i in range(D // 8):
            pltpu.make_async_copy(
                t_v.at[slot, i], out_hbm.at[0, i, j], wsem
            ).wait()

    def transpose(gslot, wslot, l):
        for k in range(8):
            row = 16 * k + lanes
            for i in range(D // 8):
                for s in range(8):
                    col = jnp.full((16,), 8 * i + s, jnp.int32)
                    t_v[wslot, i, s, 16 * k:16 * (k + 1)] = plsc.load_gather(
                        rows_v.at[gslot], [row, col]
                    )

    for slot in range(2):
        for i in range(D // 8):
            for s in range(8):
                for k in range(8):
                    t_v[slot, i, s, 16 * k:16 * (k + 1)] = zv

    pltpu.sync_copy(idxt_hbm.at[:, pl.ds(128 * j, 128)], idx_all)

    fire_writes(0, 0)
    fire_writes(1, LP - 1)
    fire_gather(0, 0)
    fire_gather(1, 1)

    def step(l, gslot, wslot, prefetch):
        wait_gather(gslot)
        if prefetch:
            fire_gather((gslot + 2) % 3, l + 2)
        drain_writes(wslot)
        transpose(gslot, wslot, l)
        fire_writes(wslot, l + 1)

    @pl.loop(0, L - 2, step=6)
    def _pos(base):
        for b in range(6):
            step(base + b, b % 3, b % 2, True)

    step(L - 2, (L - 2) % 3, 0, False)
    step(L - 1, (L - 1) % 3, 1, False)

    drain_writes(0)
    drain_writes(1)


_CB = 8192
_NBLK = (VOCAB + _CB - 1) // _CB
_RPB = _CB // 4


def _tc_transpose_body(in_ref, out_ref):
    x = in_ref[...]
    y = x.T
    out_ref[...] = jnp.concatenate(
        [y[q * _RPB:(q + 1) * _RPB, :] for q in range(4)], axis=1
    )


_tc_transpose = pl.pallas_call(
    _tc_transpose_body,
    out_shape=jax.ShapeDtypeStruct((_NBLK * _RPB, 4 * D), jnp.float32),
    grid=(_NBLK,),
    in_specs=[pl.BlockSpec((D, _CB), lambda g: (0, g))],
    out_specs=pl.BlockSpec((_RPB, 4 * D), lambda g: (g, 0)),
)


def kernel(indices, table):
    idxt = indices.astype(jnp.int32).T
    table128 = _tc_transpose(table.T)
    table32 = table128.reshape(_NBLK * _CB, D)
    out5 = _emb_lookup(idxt, table32)
    val_emb = out5.transpose(2, 4, 0, 1, 3).reshape(B, LP, D)
    val_len = jnp.full((B,), LP, dtype=jnp.int64)
    return (val_emb, val_len)


# speedup vs baseline: 1.1785x; 1.1777x over previous
import functools

import jax
import jax.numpy as jnp
from jax import lax
from jax.experimental import pallas as pl
from jax.experimental.pallas import tpu as pltpu
from jax.experimental.pallas import tpu_sc as plsc

B = 4096
L = 50
VOCAB = 1000000
D = 32
LP = L + 2

NC = 2
NS = 16
NW = NC * NS
BT = B // 128

_mesh = plsc.VectorSubcoreMesh(core_axis_name="c", subcore_axis_name="s")


@functools.partial(
    pl.kernel,
    out_type=jax.ShapeDtypeStruct((LP, D // 8, BT, 8, 128), jnp.float32),
    mesh=_mesh,
    scratch_types=[
        pltpu.VMEM((L, 128), jnp.int32),
        pltpu.VMEM((3, 128), jnp.int32),
        pltpu.VMEM((3, 128, D), jnp.float32),
        pltpu.VMEM((2, D // 8, 8, 128), jnp.float32),
        pltpu.SemaphoreType.DMA,
        pltpu.SemaphoreType.DMA,
    ],
    compiler_params=pltpu.CompilerParams(
        use_tc_tiling_on_sc=False, needs_layout_passes=False
    ),
)
def _emb_lookup(idxt_hbm, table_hbm, out_hbm, idx_all, idxq_v, rows_v, t_v,
                gsem, wsem):
    j = lax.axis_index("s") * NC + lax.axis_index("c")
    lanes = lax.iota(jnp.int32, 16)
    zv = jnp.zeros((16,), jnp.float32)

    def fire_gather(slot, l):
        for k in range(8):
            v = idx_all[l, 16 * k:16 * (k + 1)]
            idxq_v[slot, 16 * k:16 * (k + 1)] = (
                ((v >> 13) << 13) | ((v & 2047) << 2) | ((v >> 11) & 3)
            )
        pltpu.async_copy(
            table_hbm.at[idxq_v.at[slot]], rows_v.at[slot], gsem
        )

    def wait_gather(slot):
        pltpu.make_async_copy(
            table_hbm.at[pl.ds(0, 128), :], rows_v.at[slot], gsem
        ).wait()

    def fire_writes(slot, lo):
        for i in range(D // 8):
            pltpu.async_copy(t_v.at[slot, i], out_hbm.at[lo, i, j], wsem)

    def drain_writes(slot):
        for i in range(D // 8):
            pltpu.make_async_copy(
                t_v.at[slot, i], out_hbm.at[0, i, j], wsem
            ).wait()

    def transpose(gslot, wslot, l):
        for k in range(8):
            row = 16 * k + lanes
            vals = [
                plsc.load_gather(
                    rows_v.at[gslot],
                    [row, jnp.full((16,), d, jnp.int32)],
                )
                for d in range(D)
            ]
            for i in range(D // 8):
                for s in range(8):
                    t_v[wslot, i, s, 16 * k:16 * (k + 1)] = vals[8 * i + s]

    for slot in range(2):
        for i in range(D // 8):
            for s in range(8):
                for k in range(8):
                    t_v[slot, i, s, 16 * k:16 * (k + 1)] = zv

    pltpu.sync_copy(idxt_hbm.at[:, pl.ds(128 * j, 128)], idx_all)

    fire_writes(0, 0)
    fire_writes(1, LP - 1)
    fire_gather(0, 0)
    fire_gather(1, 1)

    def step(l, gslot, wslot, prefetch):
        wait_gather(gslot)
        if prefetch:
            fire_gather((gslot + 2) % 3, l + 2)
        drain_writes(wslot)
        transpose(gslot, wslot, l)
        fire_writes(wslot, l + 1)

    @pl.loop(0, L - 2, step=6)
    def _pos(base):
        for b in range(6):
            step(base + b, b % 3, b % 2, True)

    step(L - 2, (L - 2) % 3, 0, False)
    step(L - 1, (L - 1) % 3, 1, False)

    drain_writes(0)
    drain_writes(1)


_CB = 8192
_NBLK = (VOCAB + _CB - 1) // _CB
_RPB = _CB // 4


def _tc_transpose_body(in_ref, out_ref):
    x = in_ref[...]
    y = x.T
    out_ref[...] = jnp.concatenate(
        [y[q * _RPB:(q + 1) * _RPB, :] for q in range(4)], axis=1
    )


_tc_transpose = pl.pallas_call(
    _tc_transpose_body,
    out_shape=jax.ShapeDtypeStruct((_NBLK * _RPB, 4 * D), jnp.float32),
    grid=(_NBLK,),
    in_specs=[pl.BlockSpec((D, _CB), lambda g: (0, g))],
    out_specs=pl.BlockSpec((_RPB, 4 * D), lambda g: (g, 0)),
)


def kernel(indices, table):
    idxt = indices.astype(jnp.int32).T
    table128 = _tc_transpose(table.T)
    table32 = table128.reshape(_NBLK * _CB, D)
    out5 = _emb_lookup(idxt, table32)
    val_emb = out5.transpose(2, 4, 0, 1, 3).reshape(B, LP, D)
    val_len = jnp.full((B,), LP, dtype=jnp.int64)
    return (val_emb, val_len)
